# baseline (speedup 1.0000x reference)
import functools

import jax
import jax.numpy as jnp
from jax import lax
from jax.experimental import pallas as pl
from jax.experimental.pallas import tpu as pltpu
from jax.experimental.pallas import tpu_sc as plsc

B = 4096
N_DENSE = 13
N_SPARSE = 26
VOCAB = 100000
D = 16

NUM_CORES = 2
NUM_SUBCORES = 16
NUM_WORKERS = NUM_CORES * NUM_SUBCORES
N_HALF = 2
F_PER_W = N_SPARSE // N_HALF


def _sc_gather(vt_flat, table_t):
    mesh = plsc.VectorSubcoreMesh(core_axis_name="c", subcore_axis_name="s")

    @functools.partial(
        pl.kernel,
        mesh=mesh,
        out_type=jax.ShapeDtypeStruct((2 * D * N_HALF * B,), jnp.float32),
        compiler_params=pltpu.CompilerParams(needs_layout_passes=False),
        scratch_types=[
            pltpu.VMEM((VOCAB,), jnp.float32),
            pltpu.VMEM((B,), jnp.int32),
            pltpu.VMEM((B,), jnp.float32),
            pltpu.VMEM((B,), jnp.float32),
            pltpu.SemaphoreType.DMA,
            pltpu.SemaphoreType.DMA,
        ],
    )
    def body(vt_hbm, table_hbm, p_hbm, row_v, idx_v, s_v, q_v, semr, semi):
        cid = lax.axis_index("c")
        sid = lax.axis_index("s")
        wid = sid * NUM_CORES + cid
        d = wid // N_HALF
        half = wid % N_HALF

        for k in range(F_PER_W):
            f = half * F_PER_W + k
            r = f * D + d
            rcp = pltpu.make_async_copy(table_hbm.at[r], row_v, semr)
            rcp.start()
            icp = pltpu.make_async_copy(vt_hbm.at[pl.ds(f * B, B)],
                                        idx_v, semi)
            icp.start()
            icp.wait()
            rcp.wait()

            if k == 0:
                def gather0(i, carry):
                    sl = pl.ds(i * 16, 16)
                    v = plsc.load_gather(row_v, [idx_v[sl]])
                    s_v[sl] = v
                    q_v[sl] = v * v
                    return carry
                lax.fori_loop(0, B // 16, gather0, 0)
            else:
                def gatheracc(i, carry):
                    sl = pl.ds(i * 16, 16)
                    v = plsc.load_gather(row_v, [idx_v[sl]])
                    s_v[sl] = s_v[sl] + v
                    q_v[sl] = q_v[sl] + v * v
                    return carry
                lax.fori_loop(0, B // 16, gatheracc, 0)

        off = (d * N_HALF + half) * B
        pltpu.sync_copy(s_v, p_hbm.at[pl.ds(off, B)])
        pltpu.sync_copy(q_v, p_hbm.at[pl.ds(D * N_HALF * B + off, B)])

    return body(vt_flat, table_t)


def _tc_head(p, wo_col, bo):
    def body(p_ref, wo_ref, bo_ref, out_ref):
        pb = p_ref[...]
        s = pb[0, :, 0, :] + pb[0, :, 1, :]
        q = pb[1, :, 0, :] + pb[1, :, 1, :]
        att = 0.5 * (s * s - q)
        logit = jnp.sum(att * wo_ref[...], axis=0, keepdims=True)
        out_ref[...] = jax.nn.sigmoid(logit + bo_ref[...])

    return pl.pallas_call(
        body,
        out_shape=jax.ShapeDtypeStruct((1, B), jnp.float32),
    )(p, wo_col, bo)


def kernel(inputs, emb_tables, W1, b1, W2, b2, Wo, bo):
    del W1, b1, W2, b2
    vt_flat = inputs[:, N_DENSE:].T.reshape(-1)
    table_t = jnp.transpose(emb_tables, (0, 2, 1)).reshape(N_SPARSE * D, VOCAB)
    p = _sc_gather(vt_flat, table_t).reshape(2, D, N_HALF, B)
    out = _tc_head(p, Wo.reshape(D, 1), bo.reshape(1, 1))
    return out.reshape(B, 1)


# speedup vs baseline: 1.0934x; 1.0934x over previous
import functools

import jax
import jax.numpy as jnp
from jax import lax
from jax.experimental import pallas as pl
from jax.experimental.pallas import tpu as pltpu
from jax.experimental.pallas import tpu_sc as plsc

B = 4096
N_DENSE = 13
N_SPARSE = 26
VOCAB = 100000
D = 16

NUM_CORES = 2
NUM_SUBCORES = 16
NUM_WORKERS = NUM_CORES * NUM_SUBCORES
N_HALF = 2
F_PER_W = N_SPARSE // N_HALF

SPLIT = 49920
TPAD = 1024
TAILN = 256
TAIL0 = VOCAB - TAILN
H2LEN = VOCAB - SPLIT
TAILDST = TAIL0 - SPLIT
H1LEN = TAILDST + TPAD


def _sc_gather(vt_flat, table_t, table_tail):
    mesh = plsc.VectorSubcoreMesh(core_axis_name="c", subcore_axis_name="s")

    @functools.partial(
        pl.kernel,
        mesh=mesh,
        out_type=jax.ShapeDtypeStruct((2 * D * N_HALF * B,), jnp.float32),
        compiler_params=pltpu.CompilerParams(needs_layout_passes=False),
        scratch_types=[
            pltpu.VMEM((SPLIT,), jnp.float32),
            pltpu.VMEM((H1LEN,), jnp.float32),
            pltpu.VMEM((B,), jnp.int32),
            pltpu.VMEM((B,), jnp.int32),
            pltpu.VMEM((B,), jnp.float32),
            pltpu.VMEM((B,), jnp.float32),
            pltpu.SemaphoreType.DMA,
            pltpu.SemaphoreType.DMA,
            pltpu.SemaphoreType.DMA,
            pltpu.SemaphoreType.DMA,
        ],
    )
    def body(vt_hbm, table_hbm, tail_hbm, p_hbm,
             h0_v, h1_v, idx0_v, idx1_v, s_v, q_v,
             sema, semb, semt, semi):
        cid = lax.axis_index("c")
        sid = lax.axis_index("s")
        wid = sid * NUM_CORES + cid
        d = wid // N_HALF
        half = wid % N_HALF

        def row_of(k):
            return (half * F_PER_W + k) * D + d

        def cp_a(k):
            return pltpu.make_async_copy(
                table_hbm.at[row_of(k)].at[pl.ds(0, SPLIT)], h0_v, sema)

        def cp_b(k):
            return pltpu.make_async_copy(
                table_hbm.at[row_of(k)].at[pl.ds(SPLIT, SPLIT)],
                h1_v.at[pl.ds(0, SPLIT)], semb)

        def cp_t(k):
            return pltpu.make_async_copy(
                tail_hbm.at[pl.ds(row_of(k) * TPAD, TPAD)],
                h1_v.at[pl.ds(TAILDST, TPAD)], semt)

        def cp_i(k, ib):
            f = half * F_PER_W + k
            return pltpu.make_async_copy(vt_hbm.at[pl.ds(f * B, B)], ib, semi)

        def masked_pass(buf, ib, base, length, first):
            def step(i, carry):
                sl = pl.ds(i * 16, 16)
                idx = ib[sl]
                local = idx - base
                valid = (local >= 0) & (local < length)
                safe = jnp.where(valid, local, 0)
                v = plsc.load_gather(buf, [safe])
                vm = jnp.where(valid, v, jnp.zeros((16,), jnp.float32))
                if first:
                    s_v[sl] = vm
                    q_v[sl] = vm * vm
                else:
                    s_v[sl] = s_v[sl] + vm
                    q_v[sl] = q_v[sl] + vm * vm
                return carry
            lax.fori_loop(0, B // 16, step, 0)

        ibufs = (idx0_v, idx1_v)
        cp_a(0).start()
        cp_i(0, ibufs[0]).start()
        for k in range(F_PER_W):
            ib = ibufs[k % 2]
            cp_i(k, ib).wait()
            cp_a(k).wait()
            cp_b(k).start()
            cp_t(k).start()
            masked_pass(h0_v, ib, 0, SPLIT, first=(k == 0))
            cp_b(k).wait()
            cp_t(k).wait()
            if k + 1 < F_PER_W:
                cp_a(k + 1).start()
                cp_i(k + 1, ibufs[(k + 1) % 2]).start()
            masked_pass(h1_v, ib, SPLIT, H2LEN, first=False)

        off = (d * N_HALF + half) * B
        pltpu.sync_copy(s_v, p_hbm.at[pl.ds(off, B)])
        pltpu.sync_copy(q_v, p_hbm.at[pl.ds(D * N_HALF * B + off, B)])

    return body(vt_flat, table_t, table_tail)


def _tc_head(p, wo_col, bo):
    def body(p_ref, wo_ref, bo_ref, out_ref):
        pb = p_ref[...]
        s = pb[0, :, 0, :] + pb[0, :, 1, :]
        q = pb[1, :, 0, :] + pb[1, :, 1, :]
        att = 0.5 * (s * s - q)
        logit = jnp.sum(att * wo_ref[...], axis=0, keepdims=True)
        out_ref[...] = jax.nn.sigmoid(logit + bo_ref[...])

    return pl.pallas_call(
        body,
        out_shape=jax.ShapeDtypeStruct((1, B), jnp.float32),
    )(p, wo_col, bo)


def kernel(inputs, emb_tables, W1, b1, W2, b2, Wo, bo):
    del W1, b1, W2, b2
    vt_flat = inputs[:, N_DENSE:].T.reshape(-1)
    table_t = jnp.transpose(emb_tables, (0, 2, 1)).reshape(N_SPARSE * D, VOCAB)
    table_tail = jnp.pad(table_t[:, TAIL0:],
                         ((0, 0), (0, TPAD - TAILN))).reshape(-1)
    p = _sc_gather(vt_flat, table_t, table_tail).reshape(2, D, N_HALF, B)
    out = _tc_head(p, Wo.reshape(D, 1), bo.reshape(1, 1))
    return out.reshape(B, 1)
